# baseline (speedup 1.0000x reference)
import functools

import jax
import jax.numpy as jnp
from jax import lax
from jax.experimental import pallas as pl
from jax.experimental.pallas import tpu as pltpu
from jax.experimental.pallas import tpu_sc as plsc

N = 10000
E = 320000
NFEAT = 128
NHID = 256
NCLASS = 64

NCORES = 2
NSUB = 16
NWORK = NCORES * NSUB
EPW = E // NWORK
TAIL = 16
DROWS = 624
ZROWS = 208
NZ = DROWS // ZROWS
DREM = N - NSUB * DROWS


def _lane_broadcast(vec16, lane):
  idx = jnp.full((16,), lane, jnp.int32)
  dnums = lax.GatherDimensionNumbers(
      offset_dims=(), collapsed_slice_dims=(0,), start_index_map=(0,))
  return lax.gather(vec16, idx[:, None], dnums, (1,),
                    mode=lax.GatherScatterMode.PROMISE_IN_BOUNDS)


def _make_sc_spmm(width, nprob, chunk, src_cols=None, packed_out=False):
  jw = width // 16
  nch = EPW // chunk
  assert nch % 2 == 0 and EPW - nch * chunk == TAIL
  mesh = plsc.VectorSubcoreMesh(
      core_axis_name="c", subcore_axis_name="s",
      num_cores=NCORES, num_subcores=NSUB)
  if packed_out:
    out_type = jax.ShapeDtypeStruct((2 * N, nprob * width), jnp.bfloat16)
  else:
    out_type = jax.ShapeDtypeStruct((nprob * 2 * N, width), jnp.bfloat16)
  scratch = [
      pltpu.VMEM((2, chunk), jnp.int32),
      pltpu.VMEM((2, chunk), jnp.int32),
      pltpu.VMEM((chunk,), jnp.float32),
      pltpu.VMEM((chunk,), jnp.float32),
      pltpu.VMEM((chunk, width), jnp.bfloat16),
      pltpu.VMEM((chunk, width), jnp.bfloat16),
      pltpu.VMEM((chunk, width), jnp.bfloat16),
      pltpu.VMEM((chunk, width), jnp.bfloat16),
      pltpu.VMEM((chunk,), jnp.int32),
      pltpu.VMEM((chunk,), jnp.int32),
      pltpu.VMEM((2, TAIL), jnp.int32),
      pltpu.VMEM((TAIL,), jnp.float32),
      pltpu.VMEM((TAIL, width), jnp.bfloat16),
      pltpu.VMEM((TAIL, width), jnp.bfloat16),
  ]
  scratch += [pltpu.VMEM_SHARED((N, width), jnp.bfloat16)
              for _ in range(nprob)]
  scratch += [pltpu.SemaphoreType.DMA for _ in range(6)]

  @functools.partial(pl.kernel, out_type=out_type, mesh=mesh,
                     scratch_types=scratch,
                     compiler_params=pltpu.CompilerParams(
                         use_tc_tiling_on_sc=False,
                         needs_layout_passes=False))
  def k(*refs):
    ins = refs[:3 * nprob]
    out = refs[3 * nprob]
    (idxb0, idxb1, valb0, valb1, rbf0, rbf1, rff0, rff1, sidx0, sidx1,
     idxt, valt, rbft, rfft) = refs[3 * nprob + 1:3 * nprob + 15]
    accs = refs[3 * nprob + 15:3 * nprob + 15 + nprob]
    sems = refs[3 * nprob + 15 + nprob:]
    idxb = (idxb0, idxb1)
    valb = (valb0, valb1)
    rbf = (rbf0, rbf1)
    rff = (rff0, rff1)
    sidxb = (sidx0, sidx1)
    sem_pre = sems[0:2]
    sem_gat = sems[2:4]
    sem_sca = sems[4:6]

    cid = lax.axis_index("c")
    sid = lax.axis_index("s")
    wid = cid * NSUB + sid

    zero = jnp.zeros((32,), jnp.bfloat16)
    for i in range(chunk):
      for j in range(width // 32):
        rff0[i, pl.ds(j * 32, 32)] = zero
    nzf = DROWS // chunk
    zrem = DROWS % chunk
    for p in range(nprob):
      for kz in range(nzf):
        pltpu.sync_copy(rff0,
                        accs[p].at[pl.ds(sid * DROWS + kz * chunk, chunk)])
      if zrem:
        pltpu.sync_copy(rff0.at[pl.ds(0, zrem)],
                        accs[p].at[pl.ds(sid * DROWS + nzf * chunk, zrem)])

    @pl.when(sid == 0)
    def _():
      for p in range(nprob):
        pltpu.sync_copy(rff0.at[pl.ds(0, DREM)],
                        accs[p].at[pl.ds(NSUB * DROWS, DREM)])
    plsc.subcore_barrier()

    def unpack_scale_row(src_ref, dst_ref, e, vv):
      vvb = plsc.pack(vv, vv, format=plsc.PackFormat.INTERLEAVED)
      for j2 in range(width // 32):
        sl = pl.ds(j2 * 32, 32)
        dst_ref[e, sl] = src_ref[e, sl] * vvb

    def run_problem(idx_h, val_h, src_h, acc, coff):
      wbase = wid * EPW

      def src_at(idx_ref):
        if coff is None:
          return src_h.at[idx_ref]
        return src_h.at[idx_ref, pl.ds(coff, width)]

      def issue_pre(c, b):
        base = jnp.minimum(wbase + c * chunk, E - chunk)
        pltpu.async_copy(idx_h.at[:, pl.ds(base, chunk)], idxb[b], sem_pre[b])
        pltpu.async_copy(val_h.at[pl.ds(base, chunk)], valb[b], sem_pre[b])

      def wait_pre(b):
        pltpu.make_async_copy(
            idx_h.at[:, pl.ds(0, chunk)], idxb[b], sem_pre[b]).wait()
        pltpu.make_async_copy(
            val_h.at[pl.ds(0, chunk)], valb[b], sem_pre[b]).wait()

      def issue_gather(b):
        pltpu.async_copy(src_at(idxb[b].at[1]), rbf[b], sem_gat[b])

      def wait_gather(b):
        pltpu.make_async_copy(
            src_at(idxb[b].at[1]), rbf[b], sem_gat[b]).wait()

      def issue_scatter(b):
        pltpu.async_copy(rff[b], acc.at[sidxb[b]], sem_sca[b], add=True)

      def wait_scatter(b):
        pltpu.make_async_copy(rff[b], acc.at[sidxb[b]], sem_sca[b]).wait()

      def scale(b):
        for g in range(chunk // 16):
          vals16 = valb[b][pl.ds(g * 16, 16)]
          for l in range(16):
            e = g * 16 + l
            vv = _lane_broadcast(vals16, l)
            unpack_scale_row(rbf[b], rff[b], e, vv)

      def copy_sidx(b):
        for g in range(chunk // 16):
          sl = pl.ds(g * 16, 16)
          sidxb[b][sl] = idxb[b][0, sl]

      def steady(c, b, nb):
        wait_gather(b)
        wait_scatter(nb)
        wait_pre(nb)
        issue_gather(nb)
        scale(b)
        copy_sidx(b)
        issue_pre(c + 2, b)
        issue_scatter(b)

      issue_pre(0, 0)
      issue_pre(1, 1)
      wait_pre(0)
      issue_gather(0)
      wait_gather(0)
      wait_pre(1)
      issue_gather(1)
      scale(0)
      copy_sidx(0)
      issue_pre(2, 0)
      issue_scatter(0)

      @pl.loop(1, nch - 1, step=2)
      def _(c):
        steady(c, 1, 0)
        steady(c + 1, 0, 1)

      wait_gather(1)
      wait_scatter(0)
      scale(1)
      copy_sidx(1)
      issue_scatter(1)
      wait_pre(0)

      tbase = wbase + nch * chunk
      pltpu.sync_copy(idx_h.at[:, pl.ds(tbase, TAIL)], idxt)
      pltpu.sync_copy(val_h.at[pl.ds(tbase, TAIL)], valt)
      pltpu.async_copy(src_at(idxt.at[1]), rbft, sem_gat[0])
      pltpu.make_async_copy(src_at(idxt.at[1]), rbft, sem_gat[0]).wait()
      vals16 = valt[pl.ds(0, 16)]
      for l in range(16):
        vv = _lane_broadcast(vals16, l)
        unpack_scale_row(rbft, rfft, l, vv)
      pltpu.async_copy(rfft, acc.at[idxt.at[0]], sem_sca[0], add=True)
      wait_scatter(1)
      pltpu.make_async_copy(rfft, acc.at[idxt.at[0]], sem_sca[0]).wait()

    for p in range(nprob):
      idx_h, val_h, src_h = ins[3 * p:3 * p + 3]
      coff = None if src_cols is None else src_cols[p]
      run_problem(idx_h, val_h, src_h, accs[p], coff)

    def out_dst(p, r0, nr):
      if packed_out:
        return out.at[pl.ds(cid * N + r0, nr), pl.ds(p * width, width)]
      return out.at[pl.ds((2 * p + cid) * N + r0, nr)]

    plsc.subcore_barrier()
    for p in range(nprob):
      for kz in range(NZ):
        r0 = sid * DROWS + kz * ZROWS
        pltpu.sync_copy(accs[p].at[pl.ds(r0, ZROWS)], out_dst(p, r0, ZROWS))

    @pl.when(sid == 0)
    def _():
      for p in range(nprob):
        r0 = NSUB * DROWS
        pltpu.sync_copy(accs[p].at[pl.ds(r0, DREM)], out_dst(p, r0, DREM))

  return k


_sc_spmm_x = _make_sc_spmm(NFEAT, 1, 64)
_sc_spmm_cls = _make_sc_spmm(NCLASS, 2, 128,
                             packed_out=True)

_BLK = 1000
_GRID = N // _BLK


def _full(shape):
  return pl.BlockSpec(shape, lambda i: (0,) * len(shape))


def _rows(shape, off=0):
  return pl.BlockSpec(shape, lambda i, o=off: (i + o, 0))


def _tc_cast_x(x):
  def body(x_r, o_r):
    o_r[...] = x_r[...].astype(jnp.bfloat16)

  return pl.pallas_call(
      body,
      grid=(_GRID,),
      in_specs=[_rows((_BLK, NFEAT))],
      out_specs=_rows((_BLK, NFEAT)),
      out_shape=jax.ShapeDtypeStruct((N, NFEAT), jnp.bfloat16),
      compiler_params=pltpu.CompilerParams(
          dimension_semantics=("arbitrary",)),
  )(x)


def _tc_dense1(ax, x, W1, W2, Wlin, Whg, b2):
  def body(ax0_r, ax1_r, x_r, w1_r, w2_r, wl_r, whg_r, b_r, y2_r, hw_r):
    a = (ax0_r[...].astype(jnp.float32) + ax1_r[...].astype(jnp.float32))
    x1 = jnp.maximum(
        jnp.dot(a, w1_r[...], preferred_element_type=jnp.float32), 0.0)
    y2 = jnp.dot(x1, w2_r[...], preferred_element_type=jnp.float32)
    logits = jnp.dot(x_r[...], wl_r[...],
                     preferred_element_type=jnp.float32) + b_r[...]
    m = jnp.max(logits, axis=-1, keepdims=True)
    e = jnp.exp(logits - m)
    h0 = e / jnp.sum(e, axis=-1, keepdims=True)
    hw = jnp.dot(h0, whg_r[...], preferred_element_type=jnp.float32)
    hw_r[...] = hw.astype(jnp.bfloat16)
    y2_r[...] = y2.astype(jnp.bfloat16)

  return pl.pallas_call(
      body,
      grid=(_GRID,),
      in_specs=[
          _rows((_BLK, NFEAT)),
          _rows((_BLK, NFEAT), _GRID),
          _rows((_BLK, NFEAT)),
          _full((NFEAT, NHID)),
          _full((NHID, NCLASS)),
          _full((NFEAT, NCLASS)),
          _full((NCLASS, NCLASS)),
          _full((1, NCLASS)),
      ],
      out_specs=[_rows((_BLK, NCLASS)), _rows((_BLK, NCLASS))],
      out_shape=[
          jax.ShapeDtypeStruct((N, NCLASS), jnp.bfloat16),
          jax.ShapeDtypeStruct((N, NCLASS), jnp.bfloat16),
      ],
      compiler_params=pltpu.CompilerParams(
          dimension_semantics=("arbitrary",)),
  )(ax, ax, x, W1, W2, Wlin, Whg, b2)


def _tc_final(s):
  def body(a_r, b_r, z_r):
    a = a_r[...].astype(jnp.float32)
    bb = b_r[...].astype(jnp.float32)
    u = a[:, :NCLASS] + bb[:, :NCLASS]
    m = jnp.max(u, axis=-1, keepdims=True)
    e = jnp.exp(u - m)
    sm = e / jnp.sum(e, axis=-1, keepdims=True)
    z_r[...] = sm + a[:, NCLASS:] + bb[:, NCLASS:]

  return pl.pallas_call(
      body,
      grid=(_GRID,),
      in_specs=[_rows((_BLK, 2 * NCLASS)), _rows((_BLK, 2 * NCLASS), _GRID)],
      out_specs=_rows((_BLK, NCLASS)),
      out_shape=jax.ShapeDtypeStruct((N, NCLASS), jnp.float32),
      compiler_params=pltpu.CompilerParams(
          dimension_semantics=("arbitrary",)),
  )(s, s)


@jax.jit
def kernel(x, adj_indices, adj_values, q_indices, q_values,
           W1, W2, Whg, Wlin, b):
  ai = adj_indices.astype(jnp.int32)
  qi = q_indices.astype(jnp.int32)

  xb = _tc_cast_x(x)
  ax = _sc_spmm_x(ai, adj_values, xb)
  y2, hw = _tc_dense1(ax, x, W1, W2, Wlin, Whg, b.reshape(1, NCLASS))
  s = _sc_spmm_cls(ai, adj_values, y2,
                   qi, q_values, hw)
  return _tc_final(s)


# speedup vs baseline: 1.0611x; 1.0611x over previous
import functools

import jax
import jax.numpy as jnp
from jax import lax
from jax.experimental import pallas as pl
from jax.experimental.pallas import tpu as pltpu
from jax.experimental.pallas import tpu_sc as plsc

N = 10000
E = 320000
NFEAT = 128
NHID = 256
NCLASS = 64

NCORES = 2
NSUB = 16
NWORK = NCORES * NSUB
EPW = E // NWORK
TAIL = 16
DROWS = 624
ZROWS = 208
NZ = DROWS // ZROWS
DREM = N - NSUB * DROWS


def _lane_broadcast(vec16, lane):
  idx = jnp.full((16,), lane, jnp.int32)
  dnums = lax.GatherDimensionNumbers(
      offset_dims=(), collapsed_slice_dims=(0,), start_index_map=(0,))
  return lax.gather(vec16, idx[:, None], dnums, (1,),
                    mode=lax.GatherScatterMode.PROMISE_IN_BOUNDS)


def _make_sc_spmm(width, nprob, chunk, src_cols=None, packed_out=False):
  jw = width // 16
  nch = EPW // chunk
  assert nch % 2 == 0 and EPW - nch * chunk == TAIL
  mesh = plsc.VectorSubcoreMesh(
      core_axis_name="c", subcore_axis_name="s",
      num_cores=NCORES, num_subcores=NSUB)
  if packed_out:
    out_type = jax.ShapeDtypeStruct((2 * N, nprob * width), jnp.float32)
  else:
    out_type = jax.ShapeDtypeStruct((nprob * 2 * N, width), jnp.float32)
  scratch = [
      pltpu.VMEM((2, chunk), jnp.int32),
      pltpu.VMEM((2, chunk), jnp.int32),
      pltpu.VMEM((chunk,), jnp.float32),
      pltpu.VMEM((chunk,), jnp.float32),
      pltpu.VMEM((chunk, width // 2), jnp.int32),
      pltpu.VMEM((chunk, width // 2), jnp.int32),
      pltpu.VMEM((chunk, width), jnp.float32),
      pltpu.VMEM((chunk, width), jnp.float32),
      pltpu.VMEM((chunk,), jnp.int32),
      pltpu.VMEM((chunk,), jnp.int32),
      pltpu.VMEM((2, TAIL), jnp.int32),
      pltpu.VMEM((TAIL,), jnp.float32),
      pltpu.VMEM((TAIL, width // 2), jnp.int32),
      pltpu.VMEM((TAIL, width), jnp.float32),
  ]
  scratch += [pltpu.VMEM_SHARED((N, width), jnp.float32) for _ in range(nprob)]
  scratch += [pltpu.SemaphoreType.DMA for _ in range(6)]

  @functools.partial(pl.kernel, out_type=out_type, mesh=mesh,
                     scratch_types=scratch,
                     compiler_params=pltpu.CompilerParams(
                         use_tc_tiling_on_sc=False,
                         needs_layout_passes=False))
  def k(*refs):
    ins = refs[:3 * nprob]
    out = refs[3 * nprob]
    (idxb0, idxb1, valb0, valb1, rbf0, rbf1, rff0, rff1, sidx0, sidx1,
     idxt, valt, rbft, rfft) = refs[3 * nprob + 1:3 * nprob + 15]
    accs = refs[3 * nprob + 15:3 * nprob + 15 + nprob]
    sems = refs[3 * nprob + 15 + nprob:]
    idxb = (idxb0, idxb1)
    valb = (valb0, valb1)
    rbf = (rbf0, rbf1)
    rff = (rff0, rff1)
    sidxb = (sidx0, sidx1)
    sem_pre = sems[0:2]
    sem_gat = sems[2:4]
    sem_sca = sems[4:6]

    cid = lax.axis_index("c")
    sid = lax.axis_index("s")
    wid = cid * NSUB + sid

    zero = jnp.zeros((16,), jnp.float32)
    for i in range(chunk):
      for j in range(jw):
        rff0[i, pl.ds(j * 16, 16)] = zero
    nzf = DROWS // chunk
    zrem = DROWS % chunk
    for p in range(nprob):
      for kz in range(nzf):
        pltpu.sync_copy(rff0,
                        accs[p].at[pl.ds(sid * DROWS + kz * chunk, chunk)])
      if zrem:
        pltpu.sync_copy(rff0.at[pl.ds(0, zrem)],
                        accs[p].at[pl.ds(sid * DROWS + nzf * chunk, zrem)])

    @pl.when(sid == 0)
    def _():
      for p in range(nprob):
        pltpu.sync_copy(rff0.at[pl.ds(0, DREM)],
                        accs[p].at[pl.ds(NSUB * DROWS, DREM)])
    plsc.subcore_barrier()

    shift16 = jnp.full((16,), 16, jnp.int32)
    mask_hi = jnp.full((16,), -65536, jnp.int32)

    def unpack_scale_row(src_ref, dst_ref, e, vv):
      for j2 in range(width // 32):
        iv = src_ref[e, pl.ds(j2 * 16, 16)]
        ev = plsc.bitcast(lax.shift_left(iv, shift16), jnp.float32)
        ov = plsc.bitcast(jnp.bitwise_and(iv, mask_hi), jnp.float32)
        dst_ref[e, pl.ds(j2 * 32, 16)] = ev * vv
        dst_ref[e, pl.ds(j2 * 32 + 16, 16)] = ov * vv

    def run_problem(idx_h, val_h, src_h, acc, coff):
      wbase = wid * EPW

      def src_at(idx_ref):
        if coff is None:
          return src_h.at[idx_ref]
        return src_h.at[idx_ref, pl.ds(coff, width)]

      def issue_pre(c, b):
        base = jnp.minimum(wbase + c * chunk, E - chunk)
        pltpu.async_copy(idx_h.at[:, pl.ds(base, chunk)], idxb[b], sem_pre[b])
        pltpu.async_copy(val_h.at[pl.ds(base, chunk)], valb[b], sem_pre[b])

      def wait_pre(b):
        pltpu.make_async_copy(
            idx_h.at[:, pl.ds(0, chunk)], idxb[b], sem_pre[b]).wait()
        pltpu.make_async_copy(
            val_h.at[pl.ds(0, chunk)], valb[b], sem_pre[b]).wait()

      def issue_gather(b):
        pltpu.async_copy(src_at(idxb[b].at[1]), rbf[b], sem_gat[b])

      def wait_gather(b):
        pltpu.make_async_copy(
            src_at(idxb[b].at[1]), rbf[b], sem_gat[b]).wait()

      def issue_scatter(b):
        pltpu.async_copy(rff[b], acc.at[sidxb[b]], sem_sca[b], add=True)

      def wait_scatter(b):
        pltpu.make_async_copy(rff[b], acc.at[sidxb[b]], sem_sca[b]).wait()

      def scale(b):
        for g in range(chunk // 16):
          vals16 = valb[b][pl.ds(g * 16, 16)]
          for l in range(16):
            e = g * 16 + l
            vv = _lane_broadcast(vals16, l)
            unpack_scale_row(rbf[b], rff[b], e, vv)

      def copy_sidx(b):
        for g in range(chunk // 16):
          sl = pl.ds(g * 16, 16)
          sidxb[b][sl] = idxb[b][0, sl]

      def steady(c, b, nb):
        wait_gather(b)
        wait_scatter(nb)
        wait_pre(nb)
        issue_gather(nb)
        scale(b)
        copy_sidx(b)
        issue_pre(c + 2, b)
        issue_scatter(b)

      issue_pre(0, 0)
      issue_pre(1, 1)
      wait_pre(0)
      issue_gather(0)
      wait_gather(0)
      wait_pre(1)
      issue_gather(1)
      scale(0)
      copy_sidx(0)
      issue_pre(2, 0)
      issue_scatter(0)

      @pl.loop(1, nch - 1, step=2)
      def _(c):
        steady(c, 1, 0)
        steady(c + 1, 0, 1)

      wait_gather(1)
      wait_scatter(0)
      scale(1)
      copy_sidx(1)
      issue_scatter(1)
      wait_pre(0)

      tbase = wbase + nch * chunk
      pltpu.sync_copy(idx_h.at[:, pl.ds(tbase, TAIL)], idxt)
      pltpu.sync_copy(val_h.at[pl.ds(tbase, TAIL)], valt)
      pltpu.async_copy(src_at(idxt.at[1]), rbft, sem_gat[0])
      pltpu.make_async_copy(src_at(idxt.at[1]), rbft, sem_gat[0]).wait()
      vals16 = valt[pl.ds(0, 16)]
      for l in range(16):
        vv = _lane_broadcast(vals16, l)
        unpack_scale_row(rbft, rfft, l, vv)
      pltpu.async_copy(rfft, acc.at[idxt.at[0]], sem_sca[0], add=True)
      wait_scatter(1)
      pltpu.make_async_copy(rfft, acc.at[idxt.at[0]], sem_sca[0]).wait()

    for p in range(nprob):
      idx_h, val_h, src_h = ins[3 * p:3 * p + 3]
      coff = None if src_cols is None else src_cols[p]
      run_problem(idx_h, val_h, src_h, accs[p], coff)

    def out_dst(p, r0, nr):
      if packed_out:
        return out.at[pl.ds(cid * N + r0, nr), pl.ds(p * width, width)]
      return out.at[pl.ds((2 * p + cid) * N + r0, nr)]

    plsc.subcore_barrier()
    for p in range(nprob):
      for kz in range(NZ):
        r0 = sid * DROWS + kz * ZROWS
        pltpu.sync_copy(accs[p].at[pl.ds(r0, ZROWS)], out_dst(p, r0, ZROWS))

    @pl.when(sid == 0)
    def _():
      for p in range(nprob):
        r0 = NSUB * DROWS
        pltpu.sync_copy(accs[p].at[pl.ds(r0, DREM)], out_dst(p, r0, DREM))

  return k


_sc_spmm_x = _make_sc_spmm(NFEAT, 1, 64)
_sc_spmm_cls = _make_sc_spmm(NCLASS, 2, 128,
                             packed_out=True)

_BLK = 1000
_GRID = N // _BLK


def _full(shape):
  return pl.BlockSpec(shape, lambda i: (0,) * len(shape))


def _rows(shape, off=0):
  return pl.BlockSpec(shape, lambda i, o=off: (i + o, 0))


def _pack_bf16_words(v):
  parts = []
  for g in range(v.shape[1] // 32):
    p0 = v[:, g * 32:g * 32 + 16].astype(jnp.bfloat16)
    p1 = v[:, g * 32 + 16:g * 32 + 32].astype(jnp.bfloat16)
    u0 = lax.bitcast_convert_type(p0, jnp.uint16).astype(jnp.uint32)
    u1 = lax.bitcast_convert_type(p1, jnp.uint16).astype(jnp.uint32)
    parts.append(u0 | (u1 << jnp.uint32(16)))
  return lax.bitcast_convert_type(jnp.concatenate(parts, axis=1), jnp.int32)


def _tc_cast_x(x):
  def body(x_r, o_r):
    o_r[...] = _pack_bf16_words(x_r[...])

  return pl.pallas_call(
      body,
      grid=(_GRID,),
      in_specs=[_rows((_BLK, NFEAT))],
      out_specs=_rows((_BLK, NFEAT // 2)),
      out_shape=jax.ShapeDtypeStruct((N, NFEAT // 2), jnp.int32),
      compiler_params=pltpu.CompilerParams(
          dimension_semantics=("arbitrary",)),
  )(x)


def _tc_dense1(ax, x, W1, W2, Wlin, Whg, b2):
  def body(ax0_r, ax1_r, x_r, w1_r, w2_r, wl_r, whg_r, b_r, y2_r, hw_r):
    a = ax0_r[...] + ax1_r[...]
    x1 = jnp.maximum(
        jnp.dot(a, w1_r[...], preferred_element_type=jnp.float32), 0.0)
    y2 = jnp.dot(x1, w2_r[...], preferred_element_type=jnp.float32)
    logits = jnp.dot(x_r[...], wl_r[...],
                     preferred_element_type=jnp.float32) + b_r[...]
    m = jnp.max(logits, axis=-1, keepdims=True)
    e = jnp.exp(logits - m)
    h0 = e / jnp.sum(e, axis=-1, keepdims=True)
    hw = jnp.dot(h0, whg_r[...], preferred_element_type=jnp.float32)
    hw_r[...] = _pack_bf16_words(hw)
    y2_r[...] = _pack_bf16_words(y2)

  return pl.pallas_call(
      body,
      grid=(_GRID,),
      in_specs=[
          _rows((_BLK, NFEAT)),
          _rows((_BLK, NFEAT), _GRID),
          _rows((_BLK, NFEAT)),
          _full((NFEAT, NHID)),
          _full((NHID, NCLASS)),
          _full((NFEAT, NCLASS)),
          _full((NCLASS, NCLASS)),
          _full((1, NCLASS)),
      ],
      out_specs=[_rows((_BLK, NCLASS // 2)), _rows((_BLK, NCLASS // 2))],
      out_shape=[
          jax.ShapeDtypeStruct((N, NCLASS // 2), jnp.int32),
          jax.ShapeDtypeStruct((N, NCLASS // 2), jnp.int32),
      ],
      compiler_params=pltpu.CompilerParams(
          dimension_semantics=("arbitrary",)),
  )(ax, ax, x, W1, W2, Wlin, Whg, b2)


def _tc_final(s):
  def body(a_r, b_r, z_r):
    a = a_r[...]
    bb = b_r[...]
    u = a[:, :NCLASS] + bb[:, :NCLASS]
    m = jnp.max(u, axis=-1, keepdims=True)
    e = jnp.exp(u - m)
    sm = e / jnp.sum(e, axis=-1, keepdims=True)
    z_r[...] = sm + a[:, NCLASS:] + bb[:, NCLASS:]

  return pl.pallas_call(
      body,
      grid=(_GRID,),
      in_specs=[_rows((_BLK, 2 * NCLASS)), _rows((_BLK, 2 * NCLASS), _GRID)],
      out_specs=_rows((_BLK, NCLASS)),
      out_shape=jax.ShapeDtypeStruct((N, NCLASS), jnp.float32),
      compiler_params=pltpu.CompilerParams(
          dimension_semantics=("arbitrary",)),
  )(s, s)


@jax.jit
def kernel(x, adj_indices, adj_values, q_indices, q_values,
           W1, W2, Whg, Wlin, b):
  ai = adj_indices.astype(jnp.int32)
  qi = q_indices.astype(jnp.int32)

  xb = _tc_cast_x(x)
  ax = _sc_spmm_x(ai, adj_values, xb)
  y2, hw = _tc_dense1(ax, x, W1, W2, Wlin, Whg, b.reshape(1, NCLASS))
  s = _sc_spmm_cls(ai, adj_values, y2,
                   qi, q_values, hw)
  return _tc_final(s)
